# Initial kernel scaffold; baseline (speedup 1.0000x reference)
import jax
import jax.numpy as jnp
from jax.experimental import pallas as pl


def kernel(loc, depot, W_init, b_init, W_nbr, b_nbr, W_fin, b_fin, W_dep, b_dep, bn_w, bn_b):
    raise NotImplementedError("write your pallas kernel here")


# speedup vs baseline: 8.6797x; 8.6797x over previous
import functools

import jax
import jax.numpy as jnp
from jax.experimental import pallas as pl

B, N, D, E = 16, 1000, 2, 128
BLK = 200
NBLK = N // BLK
K = 10
HIGH = jax.lax.Precision.HIGHEST


def _prep_kernel(x0_ref, w_init_ref, b_init_ref, w_nbr_ref, b_nbr_ref,
                 w_fin_ref, b_fin_ref, depot_ref, w_dep_ref, b_dep_ref,
                 a_ref, p_ref, c_ref, dep_ref):
    w_fin = w_fin_ref[...]
    a_ref[...] = jax.lax.dot_general(
        w_init_ref[...] - 10.0 * w_nbr_ref[...], w_fin,
        (((1,), (0,)), ((), ())), precision=HIGH,
        preferred_element_type=jnp.float32)
    bw = jax.lax.dot_general(w_nbr_ref[...], w_fin, (((1,), (0,)), ((), ())),
                             precision=HIGH, preferred_element_type=jnp.float32)
    p_ref[...] = jax.lax.dot_general(x0_ref[0], bw, (((1,), (0,)), ((), ())),
                                     precision=HIGH,
                                     preferred_element_type=jnp.float32)
    c_ref[...] = jax.lax.dot_general(
        b_init_ref[...] + 10.0 * b_nbr_ref[...], w_fin,
        (((1,), (0,)), ((), ())), precision=HIGH,
        preferred_element_type=jnp.float32) + 11.0 * b_fin_ref[...]
    dep_ref[...] = jax.lax.dot_general(
        depot_ref[...], w_dep_ref[...], (((1,), (0,)), ((), ())),
        precision=HIGH, preferred_element_type=jnp.float32) + b_dep_ref[...]


def _main_kernel(loc_blk_ref, locT_ref, a_ref, p_ref, c_ref,
                 fe_ref, stats_ref):
    b = pl.program_id(0)
    nb = pl.program_id(1)

    xq = loc_blk_ref[0]
    xall_x = locT_ref[0, 0:1, :]
    xall_y = locT_ref[0, 1:2, :]
    dx = xall_x - xq[:, 0:1]
    dy = xall_y - xq[:, 1:2]
    dist = jnp.sqrt(dx * dx + dy * dy)

    iota = jax.lax.broadcasted_iota(jnp.int32, (BLK, N), 1)
    mask = jnp.zeros((BLK, N), jnp.float32)
    for _ in range(K):
        m = jnp.min(dist, axis=1, keepdims=True)
        idx = jnp.min(jnp.where(dist == m, iota, N), axis=1, keepdims=True)
        sel = iota == idx
        mask = mask + sel.astype(jnp.float32)
        dist = jnp.where(sel, jnp.inf, dist)

    fe = (jax.lax.dot_general(xq, a_ref[...], (((1,), (0,)), ((), ())),
                              precision=HIGH,
                              preferred_element_type=jnp.float32)
          + jax.lax.dot_general(mask, p_ref[...], (((1,), (0,)), ((), ())),
                                precision=HIGH,
                                preferred_element_type=jnp.float32)
          + c_ref[...])
    fe_ref[0] = fe

    part = jnp.stack([jnp.sum(fe, axis=0), jnp.sum(fe * fe, axis=0)])

    @pl.when(jnp.logical_and(b == 0, nb == 0))
    def _():
        stats_ref[...] = part

    @pl.when(jnp.logical_or(b != 0, nb != 0))
    def _():
        stats_ref[...] += part


def _finish_kernel(fe_ref, stats_ref, bn_w_ref, bn_b_ref, dep_ref,
                   h_ref, hdep_ref, hmean_ref):
    nb = pl.program_id(1)
    n_rows = jnp.float32(B * N)
    mean = stats_ref[0:1, :] / n_rows
    var = stats_ref[1:2, :] / n_rows - mean * mean
    inv = jax.lax.rsqrt(var + 1e-5)
    scale = inv * bn_w_ref[...]
    shift = bn_b_ref[...] - mean * scale

    fe = fe_ref[0]
    normed = fe * scale + shift
    h = jnp.where(normed >= 0, normed, 0.01 * normed)
    h_ref[0] = h

    dep = dep_ref[0]
    hdep = jnp.where(dep >= 0, dep, 0.01 * dep)

    @pl.when(nb == 0)
    def _():
        hdep_ref[0] = hdep
        hmean_ref[0] = hdep + jnp.sum(h, axis=0, keepdims=True)

    @pl.when(nb != 0)
    def _():
        hmean_ref[0] += jnp.sum(h, axis=0, keepdims=True)

    @pl.when(nb == NBLK - 1)
    def _():
        hmean_ref[0] *= 1.0 / jnp.float32(N + 1)


@functools.partial(jax.jit, static_argnames=())
def kernel(loc, depot, W_init, b_init, W_nbr, b_nbr, W_fin, b_fin,
           W_dep, b_dep, bn_w, bn_b):
    f32 = jnp.float32
    locT = jnp.swapaxes(loc, 1, 2)
    depot2 = depot.reshape(B, 2)
    b_init2 = b_init.reshape(1, -1)
    b_nbr2 = b_nbr.reshape(1, -1)
    b_fin2 = b_fin.reshape(1, -1)
    b_dep2 = b_dep.reshape(1, -1)
    bn_w2 = bn_w.reshape(1, -1)
    bn_b2 = bn_b.reshape(1, -1)

    a_mat, p_mat, c_vec, dep = pl.pallas_call(
        _prep_kernel,
        out_shape=(
            jax.ShapeDtypeStruct((D, E), f32),
            jax.ShapeDtypeStruct((N, E), f32),
            jax.ShapeDtypeStruct((1, E), f32),
            jax.ShapeDtypeStruct((B, E), f32),
        ),
    )(loc[0:1], W_init, b_init2, W_nbr, b_nbr2, W_fin, b_fin2,
      depot2, W_dep, b_dep2)

    fe, stats = pl.pallas_call(
        _main_kernel,
        grid=(B, NBLK),
        in_specs=[
            pl.BlockSpec((1, BLK, D), lambda b, nb: (b, nb, 0)),
            pl.BlockSpec((1, D, N), lambda b, nb: (b, 0, 0)),
            pl.BlockSpec((D, E), lambda b, nb: (0, 0)),
            pl.BlockSpec((N, E), lambda b, nb: (0, 0)),
            pl.BlockSpec((1, E), lambda b, nb: (0, 0)),
        ],
        out_specs=(
            pl.BlockSpec((1, BLK, E), lambda b, nb: (b, nb, 0)),
            pl.BlockSpec((2, E), lambda b, nb: (0, 0)),
        ),
        out_shape=(
            jax.ShapeDtypeStruct((B, N, E), f32),
            jax.ShapeDtypeStruct((2, E), f32),
        ),
    )(loc, locT, a_mat, p_mat, c_vec)

    h_body, h_dep, h_mean = pl.pallas_call(
        _finish_kernel,
        grid=(B, NBLK),
        in_specs=[
            pl.BlockSpec((1, BLK, E), lambda b, nb: (b, nb, 0)),
            pl.BlockSpec((2, E), lambda b, nb: (0, 0)),
            pl.BlockSpec((1, E), lambda b, nb: (0, 0)),
            pl.BlockSpec((1, E), lambda b, nb: (0, 0)),
            pl.BlockSpec((1, 1, E), lambda b, nb: (b, 0, 0)),
        ],
        out_specs=(
            pl.BlockSpec((1, BLK, E), lambda b, nb: (b, nb, 0)),
            pl.BlockSpec((1, 1, E), lambda b, nb: (b, 0, 0)),
            pl.BlockSpec((1, 1, E), lambda b, nb: (b, 0, 0)),
        ),
        out_shape=(
            jax.ShapeDtypeStruct((B, N, E), f32),
            jax.ShapeDtypeStruct((B, 1, E), f32),
            jax.ShapeDtypeStruct((B, 1, E), f32),
        ),
    )(fe, stats, bn_w2, bn_b2, dep[:, None, :])

    h = jnp.concatenate([h_dep, h_body], axis=1)
    return (h, h_mean[:, 0, :])


# speedup vs baseline: 17.9491x; 2.0679x over previous
import functools

import jax
import jax.numpy as jnp
from jax.experimental import pallas as pl
from jax.experimental.pallas import tpu as pltpu

B, N, D, E = 16, 1000, 2, 128
K = 10
HIGH = jax.lax.Precision.HIGHEST


def _fused_kernel(loc_ref, locT_ref, w_init_ref, b_init_ref, w_nbr_ref,
                  b_nbr_ref, w_fin_ref, b_fin_ref, depot_ref, w_dep_ref,
                  b_dep_ref, bn_w_ref, bn_b_ref,
                  h_ref, hmean_ref,
                  fe_s, a_s, phi_s, plo_s, c_s, dep_s, stats_s):
    p = pl.program_id(0)
    b = pl.program_id(1)

    @pl.when(jnp.logical_and(p == 0, b == 0))
    def _prep():
        w_fin = w_fin_ref[...]
        a_s[...] = jax.lax.dot_general(
            w_init_ref[...] - 10.0 * w_nbr_ref[...], w_fin,
            (((1,), (0,)), ((), ())), precision=HIGH,
            preferred_element_type=jnp.float32)
        bw = jax.lax.dot_general(
            w_nbr_ref[...], w_fin, (((1,), (0,)), ((), ())),
            precision=HIGH, preferred_element_type=jnp.float32)
        pm = jax.lax.dot_general(
            loc_ref[0], bw, (((1,), (0,)), ((), ())),
            precision=HIGH, preferred_element_type=jnp.float32)
        phi = pm.astype(jnp.bfloat16)
        phi_s[...] = phi
        plo_s[...] = (pm - phi.astype(jnp.float32)).astype(jnp.bfloat16)
        c_s[...] = jax.lax.dot_general(
            b_init_ref[...] + 10.0 * b_nbr_ref[...], w_fin,
            (((1,), (0,)), ((), ())), precision=HIGH,
            preferred_element_type=jnp.float32) + 11.0 * b_fin_ref[...]
        dep_s[...] = jax.lax.dot_general(
            depot_ref[...], w_dep_ref[...], (((1,), (0,)), ((), ())),
            precision=HIGH,
            preferred_element_type=jnp.float32) + b_dep_ref[...]

    @pl.when(p == 0)
    def _main():
        xq = loc_ref[0]
        xqT = locT_ref[0]
        dx = xq[:, 0:1] - xqT[0:1, :]
        dy = xq[:, 1:2] - xqT[1:2, :]
        dist = jnp.sqrt(dx * dx + dy * dy)

        row_i = jax.lax.broadcasted_iota(jnp.int32, (N, N), 0)
        col_i = jax.lax.broadcasted_iota(jnp.int32, (N, N), 1)
        dist = jnp.where(row_i == col_i, jnp.inf, dist)
        iota_f = row_i.astype(jnp.float32)
        big = jnp.float32(2e9)
        for _ in range(K - 1):
            m = jnp.min(dist, axis=0, keepdims=True)
            idx = jnp.min(jnp.where(dist == m, iota_f, big), axis=0,
                          keepdims=True)
            dist = jnp.where(iota_f == idx, jnp.inf, dist)
        mask = (dist == jnp.inf).astype(jnp.bfloat16)

        fe = (jax.lax.dot_general(xq, a_s[...], (((1,), (0,)), ((), ())),
                                  precision=HIGH,
                                  preferred_element_type=jnp.float32)
              + jax.lax.dot_general(mask, phi_s[...],
                                    (((0,), (0,)), ((), ())),
                                    preferred_element_type=jnp.float32)
              + jax.lax.dot_general(mask, plo_s[...],
                                    (((0,), (0,)), ((), ())),
                                    preferred_element_type=jnp.float32)
              + c_s[...])
        fe_s[b] = fe

        part = jnp.stack([jnp.sum(fe, axis=0), jnp.sum(fe * fe, axis=0)])

        @pl.when(b == 0)
        def _():
            stats_s[...] = part

        @pl.when(b != 0)
        def _():
            stats_s[...] += part

    @pl.when(p == 1)
    def _finish():
        n_rows = jnp.float32(B * N)
        mean = stats_s[0:1, :] / n_rows
        var = stats_s[1:2, :] / n_rows - mean * mean
        inv = jax.lax.rsqrt(var + 1e-5)
        scale = inv * bn_w_ref[...]
        shift = bn_b_ref[...] - mean * scale

        fe = fe_s[b]
        normed = fe * scale + shift
        h = jnp.where(normed >= 0, normed, 0.01 * normed)

        dep = dep_s[pl.ds(b, 1), :]
        hdep = jnp.where(dep >= 0, dep, 0.01 * dep)
        h_ref[0, 0:1, :] = hdep
        h_ref[0, pl.ds(1, N), :] = h
        hmean_ref[0] = (hdep + jnp.sum(h, axis=0, keepdims=True)) * (
            1.0 / jnp.float32(N + 1))


@functools.partial(jax.jit, static_argnames=())
def kernel(loc, depot, W_init, b_init, W_nbr, b_nbr, W_fin, b_fin,
           W_dep, b_dep, bn_w, bn_b):
    f32 = jnp.float32
    locT = jnp.swapaxes(loc, 1, 2)
    depot2 = depot.reshape(B, 2)
    b_init2 = b_init.reshape(1, -1)
    b_nbr2 = b_nbr.reshape(1, -1)
    b_fin2 = b_fin.reshape(1, -1)
    b_dep2 = b_dep.reshape(1, -1)
    bn_w2 = bn_w.reshape(1, -1)
    bn_b2 = bn_b.reshape(1, -1)

    const = lambda p, b: (0, 0)
    h, h_mean = pl.pallas_call(
        _fused_kernel,
        grid=(2, B),
        in_specs=[
            pl.BlockSpec((1, N, D), lambda p, b: (b, 0, 0)),
            pl.BlockSpec((1, D, N), lambda p, b: (b, 0, 0)),
            pl.BlockSpec((D, 2 * E), const),
            pl.BlockSpec((1, 2 * E), const),
            pl.BlockSpec((D, 2 * E), const),
            pl.BlockSpec((1, 2 * E), const),
            pl.BlockSpec((2 * E, E), const),
            pl.BlockSpec((1, E), const),
            pl.BlockSpec((B, D), const),
            pl.BlockSpec((D, E), const),
            pl.BlockSpec((1, E), const),
            pl.BlockSpec((1, E), const),
            pl.BlockSpec((1, E), const),
        ],
        out_specs=(
            pl.BlockSpec((1, N + 1, E), lambda p, b: (p * b, 0, 0)),
            pl.BlockSpec((1, 1, E), lambda p, b: (p * b, 0, 0)),
        ),
        out_shape=(
            jax.ShapeDtypeStruct((B, N + 1, E), f32),
            jax.ShapeDtypeStruct((B, 1, E), f32),
        ),
        scratch_shapes=[
            pltpu.VMEM((B, N, E), f32),
            pltpu.VMEM((D, E), f32),
            pltpu.VMEM((N, E), jnp.bfloat16),
            pltpu.VMEM((N, E), jnp.bfloat16),
            pltpu.VMEM((1, E), f32),
            pltpu.VMEM((B, E), f32),
            pltpu.VMEM((2, E), f32),
        ],
    )(loc, locT, W_init, b_init2, W_nbr, b_nbr2, W_fin, b_fin2,
      depot2, W_dep, b_dep2, bn_w2, bn_b2)

    return (h, h_mean[:, 0, :])
